# baseline (speedup 1.0000x reference)
import jax
import jax.numpy as jnp
from jax.experimental import pallas as pl
from jax.experimental.pallas import tpu as pltpu

_L = 128


def _conv1_kernel(c1wT, c1b, xall, o_ref):
    r = pl.program_id(1)
    hc = None
    for dy in range(2):
        for dx in range(2):
            rows = []
            for i in range(5):
                for j in range(5):
                    ry, rx = dy + i, dx + j
                    p, oy = ry % 2, ry // 2
                    s, ox = rx % 2, rx // 2
                    rows.append(xall[p, s, pl.ds(4 * r + oy, 4), ox:ox + 12])
            patches = jnp.stack(rows).reshape(25, 48 * 8 * _L)
            cand = jnp.dot(c1wT[...], patches,
                           preferred_element_type=jnp.float32)
            hc = cand if hc is None else jnp.maximum(hc, cand)
    hc = jnp.maximum(hc + c1b[...], 0.0)
    t = hc.reshape(6, 2, 2, 6, 2, 8, _L)
    o_ref[...] = t.transpose(2, 4, 0, 1, 3, 5, 6)


def _conv2_kernel(c2wT, c2b, h1, o_ref):
    hc = None
    for dy in range(2):
        for dx in range(2):
            rows = []
            for i in range(5):
                for j in range(5):
                    ry, rx = dy + i, dx + j
                    p, oy = ry % 2, ry // 2
                    s, ox = rx % 2, rx // 2
                    for ci in range(6):
                        rows.append(h1[p, s, ci, oy:oy + 4, ox:ox + 4])
            patches = jnp.stack(rows).reshape(150, 16 * 8 * _L)
            cand = jnp.dot(c2wT[...], patches,
                           preferred_element_type=jnp.float32)
            hc = cand if hc is None else jnp.maximum(hc, cand)
    hc = jnp.maximum(hc + c2b[...], 0.0)
    o_ref[...] = hc.reshape(16, 16, 8 * _L).reshape(256, 8 * _L)


def _fc_kernel(w1, b1, w2, b2, w3, b3, h2, o_ref):
    a = jnp.dot(w1[...], h2[...], preferred_element_type=jnp.float32)
    a = jnp.maximum(a + b1[...], 0.0)
    a = jnp.dot(w2[...], a, preferred_element_type=jnp.float32)
    a = jnp.maximum(a + b2[...], 0.0)
    a = jnp.dot(w3[...], a, preferred_element_type=jnp.float32)
    o_ref[...] = (a + b3[...])[None]


@jax.jit
def _forward(c1_w, c1_b, c2_w, c2_b, f1_w, f1_b, f2_w, f2_b, f3_w, f3_b, img):
    n = img.shape[0]
    m = n // _L
    tiles = n // (8 * _L)

    x = img.reshape(n, 14, 2, 14, 2).transpose(2, 4, 1, 3, 0)
    xall = x.reshape(2, 2, 14, 14, m, _L)
    c1wT, c1bc = c1_w[:, :6].T, c1_b[:, :6].T
    c2wT, c2bc = c2_w[:, :16].T, c2_b[:, :16].T
    w1 = f1_w.reshape(4, 4, _L, _L)[:, :, :16, :]
    w1 = w1.transpose(2, 0, 1, 3).reshape(256, _L).T
    w2, w3 = f2_w.T, f3_w.T
    b1, b2, b3 = f1_b.T, f2_b.T, f3_b.T

    xspec = pl.BlockSpec((2, 2, 14, 14, 8, _L),
                         lambda i, r: (0, 0, 0, 0, i, 0))

    h1 = pl.pallas_call(
        _conv1_kernel,
        out_shape=jax.ShapeDtypeStruct((2, 2, 6, 6, 6, m, _L), jnp.float32),
        grid=(tiles, 3),
        in_specs=[pl.BlockSpec((6, 25), lambda i, r: (0, 0)),
                  pl.BlockSpec((6, 1), lambda i, r: (0, 0)),
                  xspec],
        out_specs=pl.BlockSpec((2, 2, 6, 2, 6, 8, _L),
                               lambda i, r: (0, 0, 0, r, 0, i, 0)),
        compiler_params=pltpu.CompilerParams(
            dimension_semantics=("parallel", "arbitrary"),
            vmem_limit_bytes=100 * 1024 * 1024),
        cost_estimate=pl.CostEstimate(
            flops=2 * n * 86400, transcendentals=0,
            bytes_accessed=4 * (n * 784 + n * 864)),
    )(c1wT, c1bc, xall)

    h2 = pl.pallas_call(
        _conv2_kernel,
        out_shape=jax.ShapeDtypeStruct((256, n), jnp.float32),
        grid=(tiles,),
        in_specs=[
            pl.BlockSpec((16, 150), lambda i: (0, 0)),
            pl.BlockSpec((16, 1), lambda i: (0, 0)),
            pl.BlockSpec((2, 2, 6, 6, 6, 8, _L),
                         lambda i: (0, 0, 0, 0, 0, i, 0))],
        out_specs=pl.BlockSpec((256, 8 * _L), lambda i: (0, i)),
        compiler_params=pltpu.CompilerParams(
            dimension_semantics=("parallel",),
            vmem_limit_bytes=100 * 1024 * 1024),
        cost_estimate=pl.CostEstimate(
            flops=2 * n * 153600, transcendentals=0,
            bytes_accessed=4 * (n * 864 + n * 256)),
    )(c2wT, c2bc, h1)

    out = pl.pallas_call(
        _fc_kernel,
        out_shape=jax.ShapeDtypeStruct((tiles, _L, 8 * _L), jnp.float32),
        grid=(tiles,),
        in_specs=[
            pl.BlockSpec((_L, 256), lambda i: (0, 0)),
            pl.BlockSpec((_L, 1), lambda i: (0, 0)),
            pl.BlockSpec((_L, _L), lambda i: (0, 0)),
            pl.BlockSpec((_L, 1), lambda i: (0, 0)),
            pl.BlockSpec((_L, _L), lambda i: (0, 0)),
            pl.BlockSpec((_L, 1), lambda i: (0, 0)),
            pl.BlockSpec((256, 8 * _L), lambda i: (0, i)),
        ],
        out_specs=pl.BlockSpec((1, _L, 8 * _L), lambda i: (i, 0, 0)),
        compiler_params=pltpu.CompilerParams(
            dimension_semantics=("parallel",)),
        cost_estimate=pl.CostEstimate(
            flops=2 * n * 65536, transcendentals=0,
            bytes_accessed=4 * (n * 256 + n * _L)),
    )(w1, b1, w2, b2, w3, b3, h2)

    return out.transpose(0, 2, 1).reshape(n, _L)[:, :10]


def kernel(c1_w, c1_b, c2_w, c2_b, f1_w, f1_b, f2_w, f2_b, f3_w, f3_b, img):
    return _forward(c1_w, c1_b, c2_w, c2_b, f1_w, f1_b, f2_w, f2_b,
                    f3_w, f3_b, img)


# speedup vs baseline: 1.6283x; 1.6283x over previous
import jax
import jax.numpy as jnp
from jax.experimental import pallas as pl
from jax.experimental.pallas import tpu as pltpu

_L = 128
_S = 16


def _conv1_kernel(c1wT, c1b, xall, o_ref):
    r = pl.program_id(1)
    hc = None
    for dy in range(2):
        for dx in range(2):
            rows = []
            for i in range(5):
                for j in range(5):
                    ry, rx = dy + i, dx + j
                    p, oy = ry % 2, ry // 2
                    s, ox = rx % 2, rx // 2
                    rows.append(xall[p, s, pl.ds(4 * r + oy, 4), ox:ox + 12])
            patches = jnp.stack(rows).reshape(25, 48 * _S * _L)
            cand = jnp.dot(c1wT[...], patches,
                           preferred_element_type=jnp.float32)
            hc = cand if hc is None else jnp.maximum(hc, cand)
    hc = jnp.maximum(hc + c1b[...], 0.0)
    t = hc.reshape(6, 2, 2, 6, 2, _S, _L).astype(jnp.bfloat16)
    o_ref[...] = t.transpose(2, 4, 0, 1, 3, 5, 6)


def _conv2_kernel(c2wT, c2b, h1, o_ref):
    hc = None
    for dy in range(2):
        for dx in range(2):
            rows = []
            for i in range(5):
                for j in range(5):
                    ry, rx = dy + i, dx + j
                    p, oy = ry % 2, ry // 2
                    s, ox = rx % 2, rx // 2
                    for ci in range(6):
                        rows.append(h1[p, s, ci, oy:oy + 4, ox:ox + 4])
            patches = jnp.stack(rows).reshape(150, 16 * _S * _L)
            cand = jnp.dot(c2wT[...], patches,
                           preferred_element_type=jnp.float32)
            hc = cand if hc is None else jnp.maximum(hc, cand)
    hc = jnp.maximum(hc + c2b[...], 0.0)
    h2 = hc.reshape(16, 16, _S * _L).reshape(256, _S * _L)
    o_ref[...] = h2.astype(jnp.bfloat16)


def _fc_kernel(w1, b1, w2, b2, w3, b3, h2, o_ref):
    a = jnp.dot(w1[...], h2[...], preferred_element_type=jnp.float32)
    a = jnp.maximum(a + b1[...], 0.0)
    a = jnp.dot(w2[...], a, preferred_element_type=jnp.float32)
    a = jnp.maximum(a + b2[...], 0.0)
    a = jnp.dot(w3[...], a, preferred_element_type=jnp.float32)
    o_ref[...] = (a + b3[...])[None]


@jax.jit
def _forward(c1_w, c1_b, c2_w, c2_b, f1_w, f1_b, f2_w, f2_b, f3_w, f3_b, img):
    n = img.shape[0]
    m = n // _L
    tiles = n // (_S * _L)

    x = img.reshape(n, 28, 28).transpose(1, 2, 0).reshape(14, 2, 14, 2, m, _L)
    xph = [x[:, p, :, q].astype(jnp.bfloat16)
           for p in range(2) for q in range(2)]
    xall = jnp.stack(xph).reshape(2, 2, 14, 14, m, _L)
    c1wT = c1_w[:, :6].T.astype(jnp.bfloat16)
    c1bc = c1_b[:, :6].T
    c2wT = c2_w[:, :16].T.astype(jnp.bfloat16)
    c2bc = c2_b[:, :16].T
    w1 = f1_w.reshape(4, 4, _L, _L)[:, :, :16, :]
    w1 = w1.transpose(2, 0, 1, 3).reshape(256, _L).T
    w1 = w1.astype(jnp.bfloat16)
    w2, w3 = f2_w.T, f3_w.T
    b1, b2, b3 = f1_b.T, f2_b.T, f3_b.T

    xspec = pl.BlockSpec((2, 2, 14, 14, _S, _L),
                         lambda i, r: (0, 0, 0, 0, i, 0))

    h1 = pl.pallas_call(
        _conv1_kernel,
        out_shape=jax.ShapeDtypeStruct((2, 2, 6, 6, 6, m, _L),
                                       jnp.bfloat16),
        grid=(tiles, 3),
        in_specs=[pl.BlockSpec((6, 25), lambda i, r: (0, 0)),
                  pl.BlockSpec((6, 1), lambda i, r: (0, 0)),
                  xspec],
        out_specs=pl.BlockSpec((2, 2, 6, 2, 6, _S, _L),
                               lambda i, r: (0, 0, 0, r, 0, i, 0)),
        compiler_params=pltpu.CompilerParams(
            dimension_semantics=("parallel", "arbitrary"),
            vmem_limit_bytes=100 * 1024 * 1024),
        cost_estimate=pl.CostEstimate(
            flops=2 * n * 86400, transcendentals=0,
            bytes_accessed=2 * (n * 784 + n * 864)),
    )(c1wT, c1bc, xall)

    h2 = pl.pallas_call(
        _conv2_kernel,
        out_shape=jax.ShapeDtypeStruct((256, n), jnp.bfloat16),
        grid=(tiles,),
        in_specs=[
            pl.BlockSpec((16, 150), lambda i: (0, 0)),
            pl.BlockSpec((16, 1), lambda i: (0, 0)),
            pl.BlockSpec((2, 2, 6, 6, 6, _S, _L),
                         lambda i: (0, 0, 0, 0, 0, i, 0))],
        out_specs=pl.BlockSpec((256, _S * _L), lambda i: (0, i)),
        compiler_params=pltpu.CompilerParams(
            dimension_semantics=("parallel",),
            vmem_limit_bytes=100 * 1024 * 1024),
        cost_estimate=pl.CostEstimate(
            flops=2 * n * 153600, transcendentals=0,
            bytes_accessed=2 * (n * 864 + n * 256)),
    )(c2wT, c2bc, h1)

    out = pl.pallas_call(
        _fc_kernel,
        out_shape=jax.ShapeDtypeStruct((tiles, _L, _S * _L), jnp.float32),
        grid=(tiles,),
        in_specs=[
            pl.BlockSpec((_L, 256), lambda i: (0, 0)),
            pl.BlockSpec((_L, 1), lambda i: (0, 0)),
            pl.BlockSpec((_L, _L), lambda i: (0, 0)),
            pl.BlockSpec((_L, 1), lambda i: (0, 0)),
            pl.BlockSpec((_L, _L), lambda i: (0, 0)),
            pl.BlockSpec((_L, 1), lambda i: (0, 0)),
            pl.BlockSpec((256, _S * _L), lambda i: (0, i)),
        ],
        out_specs=pl.BlockSpec((1, _L, _S * _L), lambda i: (i, 0, 0)),
        compiler_params=pltpu.CompilerParams(
            dimension_semantics=("parallel",)),
        cost_estimate=pl.CostEstimate(
            flops=2 * n * 65536, transcendentals=0,
            bytes_accessed=4 * (n * 256 + n * _L)),
    )(w1, b1, w2, b2, w3, b3, h2)

    return out.transpose(0, 2, 1).reshape(n, _L)[:, :10]


def kernel(c1_w, c1_b, c2_w, c2_b, f1_w, f1_b, f2_w, f2_b, f3_w, f3_b, img):
    return _forward(c1_w, c1_b, c2_w, c2_b, f1_w, f1_b, f2_w, f2_b,
                    f3_w, f3_b, img)
